# baseline (speedup 1.0000x reference)
import jax
import jax.numpy as jnp
from jax import lax
from jax.experimental import pallas as pl
from jax.experimental.pallas import tpu as pltpu

_C = 96


def _mask_body(x_ref, p_ref, acc_ref):
    i = pl.program_id(0)
    n = pl.num_programs(0)

    @pl.when(i == 0)
    def _init():
        acc_ref[...] = jnp.zeros_like(acc_ref)

    nz = (x_ref[...] != 0.0).astype(jnp.float32)
    acc_ref[...] = jnp.maximum(acc_ref[...], jnp.max(nz, axis=0, keepdims=True))

    @pl.when(i == n - 1)
    def _finalize():
        cm = acc_ref[...]
        row = lax.broadcasted_iota(jnp.int32, (_C, _C), 0)
        col = lax.broadcasted_iota(jnp.int32, (_C, _C), 1)
        tri = (col <= row).astype(jnp.float32)
        rank_inc = jnp.sum(tri * cm, axis=1, keepdims=True)
        diag = (col == row).astype(jnp.float32)
        keep_col = jnp.sum(diag * cm, axis=1, keepdims=True)
        k = jnp.sum(cm)
        rank = rank_inc - 1.0
        colf = col.astype(jnp.float32)
        placed = jnp.where(rank == colf, 1.0, 0.0) * keep_col
        pad = jnp.where((row == 0) & (colf >= k), 1.0, 0.0)
        p_ref[...] = placed + pad


def _gather_body(p_ref, x_ref, o_ref):
    o_ref[...] = jnp.dot(x_ref[...], p_ref[...],
                         preferred_element_type=jnp.float32)


def kernel(x_inaux, x_outaux):
    xi = x_inaux.reshape(-1, _C)
    xo = x_outaux.reshape(-1, _C)

    bm = 4096
    perm = pl.pallas_call(
        _mask_body,
        grid=(xi.shape[0] // bm,),
        in_specs=[pl.BlockSpec((bm, _C), lambda i: (i, 0))],
        out_specs=pl.BlockSpec((_C, _C), lambda i: (0, 0)),
        out_shape=jax.ShapeDtypeStruct((_C, _C), jnp.float32),
        scratch_shapes=[pltpu.VMEM((1, _C), jnp.float32)],
        compiler_params=pltpu.CompilerParams(
            dimension_semantics=("arbitrary",)),
    )(xi)

    bg = 4096
    out = pl.pallas_call(
        _gather_body,
        grid=(xo.shape[0] // bg,),
        in_specs=[pl.BlockSpec((_C, _C), lambda i: (0, 0)),
                  pl.BlockSpec((bg, _C), lambda i: (i, 0))],
        out_specs=pl.BlockSpec((bg, _C), lambda i: (i, 0)),
        out_shape=jax.ShapeDtypeStruct(xo.shape, jnp.float32),
        compiler_params=pltpu.CompilerParams(
            dimension_semantics=("parallel",)),
    )(perm, xo)

    return out.reshape(x_outaux.shape)


# speedup vs baseline: 2.1749x; 2.1749x over previous
import jax
import jax.numpy as jnp
from jax import lax
from jax.experimental import pallas as pl
from jax.experimental.pallas import tpu as pltpu

_C = 96


def _build_perm(cm):
    row = lax.broadcasted_iota(jnp.int32, (_C, _C), 0)
    col = lax.broadcasted_iota(jnp.int32, (_C, _C), 1)
    tri = (col <= row).astype(jnp.float32)
    rank_inc = jnp.sum(tri * cm, axis=1, keepdims=True)
    diag = (col == row).astype(jnp.float32)
    keep_col = jnp.sum(diag * cm, axis=1, keepdims=True)
    k = jnp.sum(cm)
    rank = rank_inc - 1.0
    colf = col.astype(jnp.float32)
    placed = jnp.where(rank == colf, 1.0, 0.0) * keep_col
    pad = jnp.where((row == 0) & (colf >= k), 1.0, 0.0)
    return placed + pad


def _mask_body(x_ref, p_ref, acc_ref):
    b = pl.program_id(0)
    r = pl.program_id(1)

    @pl.when((b == 0) & (r == 0))
    def _init():
        acc_ref[...] = jnp.zeros_like(acc_ref)

    nz = (x_ref[...] != 0.0).astype(jnp.float32)
    red = jnp.max(nz, axis=(0, 1, 2))
    acc_ref[...] = jnp.maximum(acc_ref[...], red.reshape(1, _C))

    @pl.when((b == pl.num_programs(0) - 1) & (r == pl.num_programs(1) - 1))
    def _finalize():
        p_ref[...] = _build_perm(acc_ref[...])


def _gather_body(p_ref, x_ref, o_ref):
    blk = x_ref[...]
    rows = blk.shape[2] * blk.shape[3]
    x2 = blk.reshape(rows, _C)
    y = jnp.dot(x2, p_ref[...], preferred_element_type=jnp.float32)
    o_ref[...] = y.reshape(blk.shape)


def kernel(x_inaux, x_outaux):
    br = 28

    perm = pl.pallas_call(
        _mask_body,
        grid=(x_inaux.shape[0], x_inaux.shape[1] // br),
        in_specs=[pl.BlockSpec((1, br, 224, _C), lambda b, r: (b, r, 0, 0))],
        out_specs=pl.BlockSpec((_C, _C), lambda b, r: (0, 0)),
        out_shape=jax.ShapeDtypeStruct((_C, _C), jnp.float32),
        scratch_shapes=[pltpu.VMEM((1, _C), jnp.float32)],
        compiler_params=pltpu.CompilerParams(
            dimension_semantics=("arbitrary", "arbitrary")),
    )(x_inaux)

    out = pl.pallas_call(
        _gather_body,
        grid=(x_outaux.shape[0], x_outaux.shape[1], x_outaux.shape[2] // br),
        in_specs=[
            pl.BlockSpec((_C, _C), lambda b, t, r: (0, 0)),
            pl.BlockSpec((1, 1, br, 224, _C),
                         lambda b, t, r: (b, t, r, 0, 0)),
        ],
        out_specs=pl.BlockSpec((1, 1, br, 224, _C),
                               lambda b, t, r: (b, t, r, 0, 0)),
        out_shape=jax.ShapeDtypeStruct(x_outaux.shape, jnp.float32),
        compiler_params=pltpu.CompilerParams(
            dimension_semantics=("parallel", "parallel", "parallel")),
    )(perm, x_outaux)

    return out


# speedup vs baseline: 2.2193x; 1.0204x over previous
import jax
import jax.numpy as jnp
from jax import lax
from jax.experimental import pallas as pl
from jax.experimental.pallas import tpu as pltpu

_C = 96


def _build_perm(cm):
    row = lax.broadcasted_iota(jnp.int32, (_C, _C), 0)
    col = lax.broadcasted_iota(jnp.int32, (_C, _C), 1)
    tri = (col <= row).astype(jnp.float32)
    rank_inc = jnp.sum(tri * cm, axis=1, keepdims=True)
    diag = (col == row).astype(jnp.float32)
    keep_col = jnp.sum(diag * cm, axis=1, keepdims=True)
    k = jnp.sum(cm)
    rank = rank_inc - 1.0
    colf = col.astype(jnp.float32)
    placed = jnp.where(rank == colf, 1.0, 0.0) * keep_col
    pad = jnp.where((row == 0) & (colf >= k), 1.0, 0.0)
    return placed + pad


def _mask_body(x_ref, p_ref, acc_ref):
    b = pl.program_id(0)
    r = pl.program_id(1)

    @pl.when((b == 0) & (r == 0))
    def _init():
        acc_ref[...] = jnp.zeros_like(acc_ref)

    nz = (x_ref[...] != 0.0).astype(jnp.float32)
    red = jnp.max(nz, axis=(0, 1, 2))
    acc_ref[...] = jnp.maximum(acc_ref[...], red.reshape(1, _C))

    @pl.when((b == pl.num_programs(0) - 1) & (r == pl.num_programs(1) - 1))
    def _finalize():
        p_ref[...] = _build_perm(acc_ref[...])


def _gather_body(p_ref, x_ref, o_ref):
    blk = x_ref[...]
    rows = blk.shape[2] * blk.shape[3]
    x2 = blk.reshape(rows, _C)
    y = jnp.dot(x2, p_ref[...], preferred_element_type=jnp.float32)
    o_ref[...] = y.reshape(blk.shape)


def kernel(x_inaux, x_outaux):
    br = 56

    perm = pl.pallas_call(
        _mask_body,
        grid=(x_inaux.shape[0], x_inaux.shape[1] // br),
        in_specs=[pl.BlockSpec((1, br, 224, _C), lambda b, r: (b, r, 0, 0))],
        out_specs=pl.BlockSpec((_C, _C), lambda b, r: (0, 0)),
        out_shape=jax.ShapeDtypeStruct((_C, _C), jnp.float32),
        scratch_shapes=[pltpu.VMEM((1, _C), jnp.float32)],
        compiler_params=pltpu.CompilerParams(
            dimension_semantics=("arbitrary", "arbitrary")),
    )(x_inaux)

    out = pl.pallas_call(
        _gather_body,
        grid=(x_outaux.shape[0], x_outaux.shape[1], x_outaux.shape[2] // br),
        in_specs=[
            pl.BlockSpec((_C, _C), lambda b, t, r: (0, 0)),
            pl.BlockSpec((1, 1, br, 224, _C),
                         lambda b, t, r: (b, t, r, 0, 0)),
        ],
        out_specs=pl.BlockSpec((1, 1, br, 224, _C),
                               lambda b, t, r: (b, t, r, 0, 0)),
        out_shape=jax.ShapeDtypeStruct(x_outaux.shape, jnp.float32),
        compiler_params=pltpu.CompilerParams(
            dimension_semantics=("parallel", "parallel", "parallel")),
    )(perm, x_outaux)

    return out


# speedup vs baseline: 8.4096x; 3.7893x over previous
import jax
import jax.numpy as jnp
from jax import lax
from jax.experimental import pallas as pl
from jax.experimental.pallas import tpu as pltpu

_C = 96
_W = 224
_BR = 28


def _build_placed(cm_col):
    cc = lax.broadcasted_iota(jnp.int32, (_C, _C), 0)
    kk = lax.broadcasted_iota(jnp.int32, (_C, _C), 1)
    tri_le = (kk <= cc).astype(jnp.float32)
    rank_inc = jnp.dot(tri_le, cm_col,
                       preferred_element_type=jnp.float32)
    total_kept = jnp.sum(cm_col)
    rank = rank_inc - 1.0
    kkf = kk.astype(jnp.float32)
    placed = jnp.where(rank == kkf, 1.0, 0.0) * cm_col
    pad = jnp.where((cc == 0) & (kkf >= total_kept), 1.0, 0.0)
    return placed + pad


def _mask_body(x_ref, p_ref, acc_ref):
    b = pl.program_id(0)
    r = pl.program_id(1)

    @pl.when((b == 0) & (r == 0))
    def _init():
        acc_ref[...] = jnp.zeros_like(acc_ref)

    nz = (x_ref[...] != 0.0).astype(jnp.float32)
    acc_ref[...] = jnp.maximum(acc_ref[...], jnp.max(nz, axis=(0, 1)))

    @pl.when((b == pl.num_programs(0) - 1) & (r == pl.num_programs(1) - 1))
    def _finalize():
        cm_col = jnp.max(acc_ref[...], axis=1, keepdims=True)
        p_ref[...] = _build_placed(cm_col)


def _gather_body(p_ref, x_ref, o_ref):
    p = p_ref[...]
    for i in range(_BR):
        o_ref[0, 0, i] = lax.dot_general(
            p, x_ref[0, 0, i],
            dimension_numbers=(((0,), (0,)), ((), ())),
            preferred_element_type=jnp.float32)


def kernel(x_inaux, x_outaux):
    xi = x_inaux.transpose(0, 1, 3, 2)
    xo = x_outaux.transpose(0, 1, 2, 4, 3)

    placed = pl.pallas_call(
        _mask_body,
        grid=(4, 224 // _BR),
        in_specs=[pl.BlockSpec((1, _BR, _C, _W), lambda b, r: (b, r, 0, 0))],
        out_specs=pl.BlockSpec((_C, _C), lambda b, r: (0, 0)),
        out_shape=jax.ShapeDtypeStruct((_C, _C), jnp.float32),
        scratch_shapes=[pltpu.VMEM((_C, _W), jnp.float32)],
        compiler_params=pltpu.CompilerParams(
            dimension_semantics=("arbitrary", "arbitrary")),
    )(xi)

    out_t = pl.pallas_call(
        _gather_body,
        grid=(4, 2, 224 // _BR),
        in_specs=[
            pl.BlockSpec((_C, _C), lambda b, t, r: (0, 0)),
            pl.BlockSpec((1, 1, _BR, _C, _W),
                         lambda b, t, r: (b, t, r, 0, 0)),
        ],
        out_specs=pl.BlockSpec((1, 1, _BR, _C, _W),
                               lambda b, t, r: (b, t, r, 0, 0)),
        out_shape=jax.ShapeDtypeStruct(xo.shape, jnp.float32),
        compiler_params=pltpu.CompilerParams(
            dimension_semantics=("parallel", "parallel", "parallel")),
    )(placed, xo)

    return out_t.transpose(0, 1, 2, 4, 3)


# speedup vs baseline: 9.3955x; 1.1172x over previous
import jax
import jax.numpy as jnp
from jax import lax
from jax.experimental import pallas as pl
from jax.experimental.pallas import tpu as pltpu

_C = 96
_W = 224
_BR = 56


def _build_placed(cm_col):
    cc = lax.broadcasted_iota(jnp.int32, (_C, _C), 0)
    kk = lax.broadcasted_iota(jnp.int32, (_C, _C), 1)
    tri_le = (kk <= cc).astype(jnp.float32)
    rank_inc = jnp.dot(tri_le, cm_col,
                       preferred_element_type=jnp.float32)
    total_kept = jnp.sum(cm_col)
    rank = rank_inc - 1.0
    kkf = kk.astype(jnp.float32)
    placed = jnp.where(rank == kkf, 1.0, 0.0) * cm_col
    pad = jnp.where((cc == 0) & (kkf >= total_kept), 1.0, 0.0)
    return placed + pad


def _mask_body(x_ref, p_ref, acc_ref):
    b = pl.program_id(0)
    r = pl.program_id(1)

    @pl.when((b == 0) & (r == 0))
    def _init():
        acc_ref[...] = jnp.zeros_like(acc_ref)

    nz = (x_ref[...] != 0.0).astype(jnp.float32)
    acc_ref[...] = jnp.maximum(acc_ref[...], jnp.max(nz, axis=(0, 1)))

    @pl.when((b == pl.num_programs(0) - 1) & (r == pl.num_programs(1) - 1))
    def _finalize():
        cm_col = jnp.max(acc_ref[...], axis=1, keepdims=True)
        p_ref[...] = _build_placed(cm_col)


def _gather_body(p_ref, x_ref, o_ref):
    p = p_ref[...]
    for i in range(_BR):
        o_ref[0, 0, i] = lax.dot_general(
            p, x_ref[0, 0, i],
            dimension_numbers=(((0,), (0,)), ((), ())),
            preferred_element_type=jnp.float32)


def kernel(x_inaux, x_outaux):
    xi = x_inaux.transpose(0, 1, 3, 2)
    xo = x_outaux.transpose(0, 1, 2, 4, 3)

    placed = pl.pallas_call(
        _mask_body,
        grid=(4, 224 // _BR),
        in_specs=[pl.BlockSpec((1, _BR, _C, _W), lambda b, r: (b, r, 0, 0))],
        out_specs=pl.BlockSpec((_C, _C), lambda b, r: (0, 0)),
        out_shape=jax.ShapeDtypeStruct((_C, _C), jnp.float32),
        scratch_shapes=[pltpu.VMEM((_C, _W), jnp.float32)],
        compiler_params=pltpu.CompilerParams(
            dimension_semantics=("arbitrary", "arbitrary")),
    )(xi)

    out_t = pl.pallas_call(
        _gather_body,
        grid=(4, 2, 224 // _BR),
        in_specs=[
            pl.BlockSpec((_C, _C), lambda b, t, r: (0, 0)),
            pl.BlockSpec((1, 1, _BR, _C, _W),
                         lambda b, t, r: (b, t, r, 0, 0)),
        ],
        out_specs=pl.BlockSpec((1, 1, _BR, _C, _W),
                               lambda b, t, r: (b, t, r, 0, 0)),
        out_shape=jax.ShapeDtypeStruct(xo.shape, jnp.float32),
        compiler_params=pltpu.CompilerParams(
            dimension_semantics=("parallel", "parallel", "parallel")),
    )(placed, xo)

    return out_t.transpose(0, 1, 2, 4, 3)


# speedup vs baseline: 9.5707x; 1.0186x over previous
import jax
import jax.numpy as jnp
from jax import lax
from jax.experimental import pallas as pl
from jax.experimental.pallas import tpu as pltpu

_C = 96
_W = 224
_BR = 112


def _build_placed(cm_col):
    cc = lax.broadcasted_iota(jnp.int32, (_C, _C), 0)
    kk = lax.broadcasted_iota(jnp.int32, (_C, _C), 1)
    tri_le = (kk <= cc).astype(jnp.float32)
    rank_inc = jnp.dot(tri_le, cm_col,
                       preferred_element_type=jnp.float32)
    total_kept = jnp.sum(cm_col)
    rank = rank_inc - 1.0
    kkf = kk.astype(jnp.float32)
    placed = jnp.where(rank == kkf, 1.0, 0.0) * cm_col
    pad = jnp.where((cc == 0) & (kkf >= total_kept), 1.0, 0.0)
    return placed + pad


def _mask_body(x_ref, p_ref, acc_ref):
    b = pl.program_id(0)
    r = pl.program_id(1)

    @pl.when((b == 0) & (r == 0))
    def _init():
        acc_ref[...] = jnp.zeros_like(acc_ref)

    nz = (x_ref[...] != 0.0).astype(jnp.float32)
    acc_ref[...] = jnp.maximum(acc_ref[...], jnp.max(nz, axis=(0, 1)))

    @pl.when((b == pl.num_programs(0) - 1) & (r == pl.num_programs(1) - 1))
    def _finalize():
        cm_col = jnp.max(acc_ref[...], axis=1, keepdims=True)
        p_ref[...] = _build_placed(cm_col)


def _gather_body(p_ref, x_ref, o_ref):
    p = p_ref[...]
    for i in range(_BR):
        o_ref[0, 0, i] = lax.dot_general(
            p, x_ref[0, 0, i],
            dimension_numbers=(((0,), (0,)), ((), ())),
            preferred_element_type=jnp.float32)


def kernel(x_inaux, x_outaux):
    xi = x_inaux.transpose(0, 1, 3, 2)
    xo = x_outaux.transpose(0, 1, 2, 4, 3)

    placed = pl.pallas_call(
        _mask_body,
        grid=(4, 224 // _BR),
        in_specs=[pl.BlockSpec((1, _BR, _C, _W), lambda b, r: (b, r, 0, 0))],
        out_specs=pl.BlockSpec((_C, _C), lambda b, r: (0, 0)),
        out_shape=jax.ShapeDtypeStruct((_C, _C), jnp.float32),
        scratch_shapes=[pltpu.VMEM((_C, _W), jnp.float32)],
        compiler_params=pltpu.CompilerParams(
            dimension_semantics=("arbitrary", "arbitrary")),
    )(xi)

    out_t = pl.pallas_call(
        _gather_body,
        grid=(4, 2, 224 // _BR),
        in_specs=[
            pl.BlockSpec((_C, _C), lambda b, t, r: (0, 0)),
            pl.BlockSpec((1, 1, _BR, _C, _W),
                         lambda b, t, r: (b, t, r, 0, 0)),
        ],
        out_specs=pl.BlockSpec((1, 1, _BR, _C, _W),
                               lambda b, t, r: (b, t, r, 0, 0)),
        out_shape=jax.ShapeDtypeStruct(xo.shape, jnp.float32),
        compiler_params=pltpu.CompilerParams(
            dimension_semantics=("parallel", "parallel", "parallel")),
    )(placed, xo)

    return out_t.transpose(0, 1, 2, 4, 3)
